# Initial kernel scaffold; baseline (speedup 1.0000x reference)
import jax
import jax.numpy as jnp
from jax.experimental import pallas as pl


def kernel(x, edge_index, W_enc1, W_enc2, W_attr1, W_attr2, W_struct1):
    raise NotImplementedError("write your pallas kernel here")


# speedup vs baseline: 2.9250x; 2.9250x over previous
import functools

import jax
import jax.numpy as jnp
from jax import lax
from jax.experimental import pallas as pl
from jax.experimental.pallas import tpu as pltpu
from jax.experimental.pallas import tpu_sc as plsc
from jax._src import config as _config

N = 10000
E = 320000
NW = 32
NPAD = 10240
B = 320
ACCR = 328
EC = 12800
NPH = 10
NIN = 10
CH = 128
RB = 2000
RBD = 200


@functools.lru_cache(maxsize=None)
def _mesh():
    return plsc.VectorSubcoreMesh(core_axis_name="c", subcore_axis_name="s",
                                  num_cores=2, num_subcores=16)


_SC_PARAMS = pltpu.CompilerParams(needs_layout_passes=False)


def _wid():
    return lax.axis_index("c") * 16 + lax.axis_index("s")


def _zero_rows(ref, nrows, ncol16):
    z16 = jnp.zeros((16,), jnp.float32)

    def zrow(i, carry):
        for j in range(ncol16):
            ref[i, pl.ds(j * 16, 16)] = z16
        return carry

    lax.fori_loop(jnp.int32(0), jnp.int32(nrows), zrow, jnp.int32(0))


@functools.lru_cache(maxsize=None)
def _make_segsum(d):
    nslab = d // 8

    def body(qT_hbm, srcg_hbm, dstloc_hbm, out_hbm, slab, idx_s, idx_d, acc):
        wid = _wid()
        _zero_rows(acc, ACCR, d // 16)

        def do_slab(s, carry):
            pltpu.sync_copy(qT_hbm.at[pl.ds(s * 8, 8)], slab)

            def do_phase(p, carry2):
                pltpu.sync_copy(srcg_hbm.at[wid, p], idx_s)
                pltpu.sync_copy(dstloc_hbm.at[wid, p], idx_d)

                def do_chunk(r, carry3):
                    for g in range(CH // 16):
                        src16 = idx_s[r, pl.ds(g * 16, 16)]
                        dl16 = idx_d[r, pl.ds(g * 16, 16)]
                        for col in range(8):
                            rowv = jnp.full((16,), col, jnp.int32)
                            v = plsc.load_gather(slab, [rowv, src16])
                            colv = (s * 8 + col) + jnp.zeros((16,), jnp.int32)
                            plsc.addupdate_scatter(acc, [dl16, colv], v)
                    return carry3

                lax.fori_loop(jnp.int32(0), jnp.int32(NIN), do_chunk,
                              jnp.int32(0))
                return carry2

            lax.fori_loop(jnp.int32(0), jnp.int32(NPH), do_phase, jnp.int32(0))
            return carry

        lax.fori_loop(jnp.int32(0), jnp.int32(nslab), do_slab, jnp.int32(0))
        pltpu.sync_copy(acc.at[pl.ds(0, B)], out_hbm.at[pl.ds(wid * B, B)])

    return pl.kernel(
        body,
        out_type=jax.ShapeDtypeStruct((NPAD, d), jnp.float32),
        mesh=_mesh(),
        name=f"sc_segsum{d}",
        compiler_params=_SC_PARAMS,
        scratch_types=[
            pltpu.VMEM((8, NPAD), jnp.float32),
            pltpu.VMEM((NIN, CH), jnp.int32),
            pltpu.VMEM((NIN, CH), jnp.int32),
            pltpu.VMEM((ACCR, d), jnp.float32),
        ],
    )


def _deg_body(dstloc_hbm, out_hbm, idx_d, acc):
    wid = _wid()
    _zero_rows(acc, ACCR, 1)
    ones16 = jnp.ones((16,), jnp.float32)
    zcol = jnp.zeros((16,), jnp.int32)

    def do_phase(p, carry):
        pltpu.sync_copy(dstloc_hbm.at[wid, p], idx_d)

        def do_chunk(r, carry2):
            for g in range(CH // 16):
                dl16 = idx_d[r, pl.ds(g * 16, 16)]
                plsc.addupdate_scatter(acc, [dl16, zcol], ones16)
            return carry2

        lax.fori_loop(jnp.int32(0), jnp.int32(NIN), do_chunk, jnp.int32(0))
        return carry

    lax.fori_loop(jnp.int32(0), jnp.int32(NPH), do_phase, jnp.int32(0))
    pltpu.sync_copy(acc.at[pl.ds(0, B)], out_hbm.at[pl.ds(wid * B, B)])


@functools.lru_cache(maxsize=None)
def _make_deg():
    return pl.kernel(
        _deg_body,
        out_type=jax.ShapeDtypeStruct((NPAD, 16), jnp.float32),
        mesh=_mesh(),
        name="sc_deg",
        compiler_params=_SC_PARAMS,
        scratch_types=[
            pltpu.VMEM((NIN, CH), jnp.int32),
            pltpu.VMEM((ACCR, 16), jnp.float32),
        ],
    )


def _corr_dots_body(s1T_hbm, srcloc_hbm, dstg_hbm, out_hbm,
                    slab, idx_a, idx_b, dots):
    wid = _wid()
    z16 = jnp.zeros((16,), jnp.float32)

    def zdots(i, carry):
        dots[pl.ds(i * 16, 16)] = z16
        return carry

    lax.fori_loop(jnp.int32(0), jnp.int32(EC // 16), zdots, jnp.int32(0))
    base = wid * B

    def do_slab(s, carry):
        pltpu.sync_copy(s1T_hbm.at[pl.ds(s * 8, 8)], slab)

        def do_phase(p, carry2):
            pltpu.sync_copy(srcloc_hbm.at[wid, p], idx_a)
            pltpu.sync_copy(dstg_hbm.at[wid, p], idx_b)

            def do_chunk(r, carry3):
                for g in range(CH // 16):
                    srcg16 = idx_a[r, pl.ds(g * 16, 16)] + base
                    dst16 = idx_b[r, pl.ds(g * 16, 16)]
                    part = jnp.zeros((16,), jnp.float32)
                    for col in range(8):
                        rowv = jnp.full((16,), col, jnp.int32)
                        a = plsc.load_gather(slab, [rowv, srcg16])
                        b = plsc.load_gather(slab, [rowv, dst16])
                        part = part + a * b
                    off = (p * NIN + r) * CH + g * 16
                    dots[pl.ds(off, 16)] = dots[pl.ds(off, 16)] + part
                return carry3

            lax.fori_loop(jnp.int32(0), jnp.int32(NIN), do_chunk, jnp.int32(0))
            return carry2

        lax.fori_loop(jnp.int32(0), jnp.int32(NPH), do_phase, jnp.int32(0))
        return carry

    lax.fori_loop(jnp.int32(0), jnp.int32(8), do_slab, jnp.int32(0))
    pltpu.sync_copy(dots, out_hbm.at[wid])


@functools.lru_cache(maxsize=None)
def _make_corr_dots():
    return pl.kernel(
        _corr_dots_body,
        out_type=jax.ShapeDtypeStruct((NW, EC), jnp.float32),
        mesh=_mesh(),
        name="sc_corr_dots",
        compiler_params=_SC_PARAMS,
        scratch_types=[
            pltpu.VMEM((8, NPAD), jnp.float32),
            pltpu.VMEM((NIN, CH), jnp.int32),
            pltpu.VMEM((NIN, CH), jnp.int32),
            pltpu.VMEM((EC,), jnp.float32),
        ],
    )


def _corr_scat_body(dots_hbm, srcloc_hbm, w_hbm, out_hbm, dots, idx_a, wbuf,
                    acc):
    wid = _wid()
    _zero_rows(acc, ACCR, 1)
    zcol = jnp.zeros((16,), jnp.int32)
    pltpu.sync_copy(dots_hbm.at[wid], dots)

    def do_phase(p, carry):
        pltpu.sync_copy(srcloc_hbm.at[wid, p], idx_a)
        pltpu.sync_copy(w_hbm.at[wid, p], wbuf)

        def do_chunk(r, carry2):
            for g in range(CH // 16):
                sl16 = idx_a[r, pl.ds(g * 16, 16)]
                w16 = wbuf[r, pl.ds(g * 16, 16)]
                off = (p * NIN + r) * CH + g * 16
                dt = dots[pl.ds(off, 16)]
                sig = 1.0 / (1.0 + jnp.exp(-dt))
                v = w16 * (1.0 - 2.0 * sig)
                plsc.addupdate_scatter(acc, [sl16, zcol], v)
            return carry2

        lax.fori_loop(jnp.int32(0), jnp.int32(NIN), do_chunk, jnp.int32(0))
        return carry

    lax.fori_loop(jnp.int32(0), jnp.int32(NPH), do_phase, jnp.int32(0))
    pltpu.sync_copy(acc.at[pl.ds(0, B)], out_hbm.at[pl.ds(wid * B, B)])


@functools.lru_cache(maxsize=None)
def _make_corr_scat():
    return pl.kernel(
        _corr_scat_body,
        out_type=jax.ShapeDtypeStruct((NPAD, 16), jnp.float32),
        mesh=_mesh(),
        name="sc_corr_scat",
        compiler_params=_SC_PARAMS,
        scratch_types=[
            pltpu.VMEM((EC,), jnp.float32),
            pltpu.VMEM((NIN, CH), jnp.int32),
            pltpu.VMEM((NIN, CH), jnp.float32),
            pltpu.VMEM((ACCR, 16), jnp.float32),
        ],
    )


def _leaky(t):
    return jnp.where(t >= 0, t, 0.01 * t)


def _tca_body(x_ref, w_ref, dg_ref, q_ref, dinv_ref):
    deg = dg_ref[:, 0:1] + 1.0
    dinv = lax.rsqrt(deg)
    p = jnp.dot(x_ref[...], w_ref[...], preferred_element_type=jnp.float32)
    q_ref[...] = p * dinv
    dinv_ref[...] = dinv


def _tcb_body(s_ref, q_ref, dinv_ref, w_ref, q2_ref):
    dinv = dinv_ref[...]
    h = jnp.maximum((s_ref[...] + q_ref[...]) * dinv, 0.0)
    q2_ref[...] = jnp.dot(h, w_ref[...],
                          preferred_element_type=jnp.float32) * dinv


def _tcc_body(s_ref, q_ref, dinv_ref, qz_ref):
    dinv = dinv_ref[...]
    z = jnp.maximum((s_ref[...] + q_ref[...]) * dinv, 0.0)
    qz_ref[...] = z * dinv


def _tcd_body(s_ref, q_ref, dinv_ref, wa_ref, ws_ref, s1_ref, q4_ref):
    dinv = dinv_ref[...]
    az = (s_ref[...] + q_ref[...]) * dinv
    a1 = _leaky(jnp.dot(az, wa_ref[...], preferred_element_type=jnp.float32))
    s1_ref[...] = _leaky(jnp.dot(az, ws_ref[...],
                                 preferred_element_type=jnp.float32))
    q4_ref[...] = a1 * dinv


def _tce_body(s_ref, q_ref, dinv_ref, w_ref, x_ref, attr_ref, ae_ref):
    dinv = dinv_ref[...]
    t = (s_ref[...] + q_ref[...]) * dinv
    attr = _leaky(jnp.dot(t, w_ref[...], preferred_element_type=jnp.float32))
    attr_ref[...] = attr
    diff = x_ref[...] - attr
    ae_ref[...] = jnp.sqrt(jnp.sum(diff * diff, axis=1, keepdims=True))


def _dense_body(s1b_ref, s1f_ref, st_ref, rq_ref):
    p = lax.dot_general(s1b_ref[...], s1f_ref[...],
                        (((1,), (1,)), ((), ())),
                        preferred_element_type=jnp.float32)
    sg = jax.nn.sigmoid(p)
    st_ref[...] = sg
    rq_ref[...] = jnp.sum(sg * sg, axis=1, keepdims=True)


def _tcf_body(rq_ref, cp_ref, ae_ref, sc_ref):
    corr = cp_ref[:, 0:1]
    se = jnp.sqrt(jnp.maximum(rq_ref[...] + corr, 0.0))
    rec = 0.5 * ae_ref[...] + 0.5 * se
    mn = jnp.min(rec)
    mx = jnp.max(rec)
    sc_ref[...] = (rec - mn) / (mx - mn)


def _partition(major, minors, pads, padrow):
    bnd = jnp.searchsorted(major, (jnp.arange(NW + 1, dtype=jnp.int32) * B)
                           .astype(major.dtype)).astype(jnp.int32)
    j = jnp.arange(EC, dtype=jnp.int32)
    g = jnp.clip(bnd[:NW, None] + j[None, :], 0, E - 1)
    valid = j[None, :] < (bnd[1:, None] - bnd[:NW, None])
    majloc = jnp.where(valid, major[g] - (jnp.arange(NW, dtype=jnp.int32)
                                          * B)[:, None], padrow)
    outs = [jnp.where(valid, m[g], pv) for m, pv in zip(minors, pads)]
    shape = (NW, NPH, NIN, CH)
    return (majloc.astype(jnp.int32).reshape(shape),
            [o.reshape(shape) for o in outs])


def kernel(x, edge_index, W_enc1, W_enc2, W_attr1, W_attr2, W_struct1):
    with _config.enable_x64(False):
        return _kernel_impl(x, edge_index, W_enc1, W_enc2, W_attr1, W_attr2,
                            W_struct1)


def _kernel_impl(x, edge_index, W_enc1, W_enc2, W_attr1, W_attr2, W_struct1):
    x = x.astype(jnp.float32)
    ei = edge_index.astype(jnp.int32)
    src, dst = ei[0], ei[1]

    k1 = src * N + dst
    ks1 = jnp.sort(k1)
    src_s = ks1 // N
    dst_s = ks1 - src_s * N
    wdup = jnp.concatenate([
        jnp.ones((1,), jnp.float32),
        (ks1[1:] != ks1[:-1]).astype(jnp.float32),
    ])
    k2 = dst * N + src
    ks2 = jnp.sort(k2)
    dst_t = ks2 // N
    src_t = ks2 - dst_t * N

    dstloc_p, (srcg_p,) = _partition(dst_t, [src_t], [jnp.int32(0)], B)
    srcloc_p, (dstg_p, w_p) = _partition(
        src_s, [dst_s, wdup], [jnp.int32(0), jnp.float32(0.0)], 0)

    degp = _make_deg()(dstloc_p)

    q1, dinv = pl.pallas_call(
        _tca_body,
        name="tc_a",
        grid=(N // RB,),
        in_specs=[
            pl.BlockSpec((RB, 128), lambda i: (i, 0)),
            pl.BlockSpec((128, 64), lambda i: (0, 0)),
            pl.BlockSpec((RB, 16), lambda i: (i, 0)),
        ],
        out_specs=[
            pl.BlockSpec((RB, 64), lambda i: (i, 0)),
            pl.BlockSpec((RB, 1), lambda i: (i, 0)),
        ],
        out_shape=[
            jax.ShapeDtypeStruct((NPAD, 64), jnp.float32),
            jax.ShapeDtypeStruct((N, 1), jnp.float32),
        ],
    )(x, W_enc1, degp)

    s1p = _make_segsum(64)(q1.T, srcg_p, dstloc_p)

    q2 = pl.pallas_call(
        _tcb_body,
        name="tc_b",
        grid=(N // RB,),
        in_specs=[
            pl.BlockSpec((RB, 64), lambda i: (i, 0)),
            pl.BlockSpec((RB, 64), lambda i: (i, 0)),
            pl.BlockSpec((RB, 1), lambda i: (i, 0)),
            pl.BlockSpec((64, 32), lambda i: (0, 0)),
        ],
        out_specs=pl.BlockSpec((RB, 32), lambda i: (i, 0)),
        out_shape=jax.ShapeDtypeStruct((NPAD, 32), jnp.float32),
    )(s1p, q1, dinv, W_enc2)

    s2p = _make_segsum(32)(q2.T, srcg_p, dstloc_p)

    qz = pl.pallas_call(
        _tcc_body,
        name="tc_c",
        grid=(N // RB,),
        in_specs=[
            pl.BlockSpec((RB, 32), lambda i: (i, 0)),
            pl.BlockSpec((RB, 32), lambda i: (i, 0)),
            pl.BlockSpec((RB, 1), lambda i: (i, 0)),
        ],
        out_specs=pl.BlockSpec((RB, 32), lambda i: (i, 0)),
        out_shape=jax.ShapeDtypeStruct((NPAD, 32), jnp.float32),
    )(s2p, q2, dinv)

    s3p = _make_segsum(32)(qz.T, srcg_p, dstloc_p)

    s1v, q4 = pl.pallas_call(
        _tcd_body,
        name="tc_d",
        grid=(N // RB,),
        in_specs=[
            pl.BlockSpec((RB, 32), lambda i: (i, 0)),
            pl.BlockSpec((RB, 32), lambda i: (i, 0)),
            pl.BlockSpec((RB, 1), lambda i: (i, 0)),
            pl.BlockSpec((32, 64), lambda i: (0, 0)),
            pl.BlockSpec((32, 64), lambda i: (0, 0)),
        ],
        out_specs=[
            pl.BlockSpec((RB, 64), lambda i: (i, 0)),
            pl.BlockSpec((RB, 64), lambda i: (i, 0)),
        ],
        out_shape=[
            jax.ShapeDtypeStruct((NPAD, 64), jnp.float32),
            jax.ShapeDtypeStruct((NPAD, 64), jnp.float32),
        ],
    )(s3p, qz, dinv, W_attr1, W_struct1)

    s4p = _make_segsum(64)(q4.T, srcg_p, dstloc_p)

    attr, attr_err = pl.pallas_call(
        _tce_body,
        name="tc_e",
        grid=(N // RB,),
        in_specs=[
            pl.BlockSpec((RB, 64), lambda i: (i, 0)),
            pl.BlockSpec((RB, 64), lambda i: (i, 0)),
            pl.BlockSpec((RB, 1), lambda i: (i, 0)),
            pl.BlockSpec((64, 128), lambda i: (0, 0)),
            pl.BlockSpec((RB, 128), lambda i: (i, 0)),
        ],
        out_specs=[
            pl.BlockSpec((RB, 128), lambda i: (i, 0)),
            pl.BlockSpec((RB, 1), lambda i: (i, 0)),
        ],
        out_shape=[
            jax.ShapeDtypeStruct((N, 128), jnp.float32),
            jax.ShapeDtypeStruct((N, 1), jnp.float32),
        ],
    )(s4p, q4, dinv, W_attr2, x)

    struct, rowsq = pl.pallas_call(
        _dense_body,
        name="tc_dense",
        grid=(N // RBD,),
        in_specs=[
            pl.BlockSpec((RBD, 64), lambda i: (i, 0)),
            pl.BlockSpec((N, 64), lambda i: (0, 0)),
        ],
        out_specs=[
            pl.BlockSpec((RBD, N), lambda i: (i, 0)),
            pl.BlockSpec((RBD, 1), lambda i: (i, 0)),
        ],
        out_shape=[
            jax.ShapeDtypeStruct((N, N), jnp.float32),
            jax.ShapeDtypeStruct((N, 1), jnp.float32),
        ],
    )(s1v, s1v)

    dotsv = _make_corr_dots()(s1v.T, srcloc_p, dstg_p)
    corrp = _make_corr_scat()(dotsv, srcloc_p, w_p)

    scores = pl.pallas_call(
        _tcf_body,
        name="tc_f",
        grid=(1,),
        in_specs=[
            pl.BlockSpec((N, 1), lambda i: (0, 0)),
            pl.BlockSpec((N, 16), lambda i: (0, 0)),
            pl.BlockSpec((N, 1), lambda i: (0, 0)),
        ],
        out_specs=pl.BlockSpec((N, 1), lambda i: (0, 0)),
        out_shape=jax.ShapeDtypeStruct((N, 1), jnp.float32),
    )(rowsq, corrp, attr_err)

    return (attr, struct, scores.reshape(N))


# speedup vs baseline: 3.0649x; 1.0478x over previous
import functools

import jax
import jax.numpy as jnp
from jax import lax
from jax.experimental import pallas as pl
from jax.experimental.pallas import tpu as pltpu
from jax.experimental.pallas import tpu_sc as plsc
from jax._src import config as _config

N = 10000
E = 320000
NW = 32
NPAD = 10240
B = 320
ACCR = 328
EC = 12800
NPH = 10
NIN = 10
CH = 128
RB = 2000
RBD = 200


@functools.lru_cache(maxsize=None)
def _mesh():
    return plsc.VectorSubcoreMesh(core_axis_name="c", subcore_axis_name="s",
                                  num_cores=2, num_subcores=16)


_SC_PARAMS = pltpu.CompilerParams(needs_layout_passes=False)


def _wid():
    return lax.axis_index("c") * 16 + lax.axis_index("s")


def _zero_rows(ref, nrows, ncol16):
    z16 = jnp.zeros((16,), jnp.float32)

    def zrow(i, carry):
        for j in range(ncol16):
            ref[i, pl.ds(j * 16, 16)] = z16
        return carry

    lax.fori_loop(jnp.int32(0), jnp.int32(nrows), zrow, jnp.int32(0))


@functools.lru_cache(maxsize=None)
def _make_segsum(d):
    nslab = d // 8

    def body(qT_hbm, srcg_hbm, dstloc_hbm, out_hbm, slab, idx_s, idx_d, acc):
        wid = _wid()
        _zero_rows(acc, ACCR, d // 16)

        def do_slab(s, carry):
            pltpu.sync_copy(qT_hbm.at[pl.ds(s * 8, 8)], slab)

            def do_phase(p, carry2):
                pltpu.sync_copy(srcg_hbm.at[wid, p], idx_s)
                pltpu.sync_copy(dstloc_hbm.at[wid, p], idx_d)

                def do_chunk(r, carry3):
                    for g in range(CH // 16):
                        src16 = idx_s[r, pl.ds(g * 16, 16)]
                        dl16 = idx_d[r, pl.ds(g * 16, 16)]
                        for col in range(8):
                            rowv = jnp.full((16,), col, jnp.int32)
                            v = plsc.load_gather(slab, [rowv, src16])
                            colv = (s * 8 + col) + jnp.zeros((16,), jnp.int32)
                            plsc.addupdate_scatter(acc, [dl16, colv], v)
                    return carry3

                lax.fori_loop(jnp.int32(0), jnp.int32(NIN), do_chunk,
                              jnp.int32(0))
                return carry2

            lax.fori_loop(jnp.int32(0), jnp.int32(NPH), do_phase, jnp.int32(0))
            return carry

        lax.fori_loop(jnp.int32(0), jnp.int32(nslab), do_slab, jnp.int32(0))
        pltpu.sync_copy(acc.at[pl.ds(0, B)], out_hbm.at[pl.ds(wid * B, B)])

    return pl.kernel(
        body,
        out_type=jax.ShapeDtypeStruct((NPAD, d), jnp.float32),
        mesh=_mesh(),
        name=f"sc_segsum{d}",
        compiler_params=_SC_PARAMS,
        scratch_types=[
            pltpu.VMEM((8, NPAD), jnp.float32),
            pltpu.VMEM((NIN, CH), jnp.int32),
            pltpu.VMEM((NIN, CH), jnp.int32),
            pltpu.VMEM((ACCR, d), jnp.float32),
        ],
    )


def _deg_body(dstloc_hbm, out_hbm, idx_d, acc):
    wid = _wid()
    _zero_rows(acc, ACCR, 1)
    ones16 = jnp.ones((16,), jnp.float32)
    zcol = jnp.zeros((16,), jnp.int32)

    def do_phase(p, carry):
        pltpu.sync_copy(dstloc_hbm.at[wid, p], idx_d)

        def do_chunk(r, carry2):
            for g in range(CH // 16):
                dl16 = idx_d[r, pl.ds(g * 16, 16)]
                plsc.addupdate_scatter(acc, [dl16, zcol], ones16)
            return carry2

        lax.fori_loop(jnp.int32(0), jnp.int32(NIN), do_chunk, jnp.int32(0))
        return carry

    lax.fori_loop(jnp.int32(0), jnp.int32(NPH), do_phase, jnp.int32(0))
    pltpu.sync_copy(acc.at[pl.ds(0, B)], out_hbm.at[pl.ds(wid * B, B)])


@functools.lru_cache(maxsize=None)
def _make_deg():
    return pl.kernel(
        _deg_body,
        out_type=jax.ShapeDtypeStruct((NPAD, 16), jnp.float32),
        mesh=_mesh(),
        name="sc_deg",
        compiler_params=_SC_PARAMS,
        scratch_types=[
            pltpu.VMEM((NIN, CH), jnp.int32),
            pltpu.VMEM((ACCR, 16), jnp.float32),
        ],
    )


def _corr_dots_body(s1T_hbm, srcloc_hbm, dstg_hbm, out_hbm,
                    slab, idx_a, idx_b, dots):
    wid = _wid()
    z16 = jnp.zeros((16,), jnp.float32)

    def zdots(i, carry):
        dots[pl.ds(i * 16, 16)] = z16
        return carry

    lax.fori_loop(jnp.int32(0), jnp.int32(EC // 16), zdots, jnp.int32(0))
    base = wid * B

    def do_slab(s, carry):
        pltpu.sync_copy(s1T_hbm.at[pl.ds(s * 8, 8)], slab)

        def do_phase(p, carry2):
            pltpu.sync_copy(srcloc_hbm.at[wid, p], idx_a)
            pltpu.sync_copy(dstg_hbm.at[wid, p], idx_b)

            def do_chunk(r, carry3):
                for g in range(CH // 16):
                    srcg16 = idx_a[r, pl.ds(g * 16, 16)] + base
                    dst16 = idx_b[r, pl.ds(g * 16, 16)]
                    part = jnp.zeros((16,), jnp.float32)
                    for col in range(8):
                        rowv = jnp.full((16,), col, jnp.int32)
                        a = plsc.load_gather(slab, [rowv, srcg16])
                        b = plsc.load_gather(slab, [rowv, dst16])
                        part = part + a * b
                    off = (p * NIN + r) * CH + g * 16
                    dots[pl.ds(off, 16)] = dots[pl.ds(off, 16)] + part
                return carry3

            lax.fori_loop(jnp.int32(0), jnp.int32(NIN), do_chunk, jnp.int32(0))
            return carry2

        lax.fori_loop(jnp.int32(0), jnp.int32(NPH), do_phase, jnp.int32(0))
        return carry

    lax.fori_loop(jnp.int32(0), jnp.int32(8), do_slab, jnp.int32(0))
    pltpu.sync_copy(dots, out_hbm.at[wid])


@functools.lru_cache(maxsize=None)
def _make_corr_dots():
    return pl.kernel(
        _corr_dots_body,
        out_type=jax.ShapeDtypeStruct((NW, EC), jnp.float32),
        mesh=_mesh(),
        name="sc_corr_dots",
        compiler_params=_SC_PARAMS,
        scratch_types=[
            pltpu.VMEM((8, NPAD), jnp.float32),
            pltpu.VMEM((NIN, CH), jnp.int32),
            pltpu.VMEM((NIN, CH), jnp.int32),
            pltpu.VMEM((EC,), jnp.float32),
        ],
    )


def _corr_scat_body(dots_hbm, srcloc_hbm, w_hbm, out_hbm, dots, idx_a, wbuf,
                    acc):
    wid = _wid()
    _zero_rows(acc, ACCR, 1)
    zcol = jnp.zeros((16,), jnp.int32)
    pltpu.sync_copy(dots_hbm.at[wid], dots)

    def do_phase(p, carry):
        pltpu.sync_copy(srcloc_hbm.at[wid, p], idx_a)
        pltpu.sync_copy(w_hbm.at[wid, p], wbuf)

        def do_chunk(r, carry2):
            for g in range(CH // 16):
                sl16 = idx_a[r, pl.ds(g * 16, 16)]
                w16 = wbuf[r, pl.ds(g * 16, 16)]
                off = (p * NIN + r) * CH + g * 16
                dt = dots[pl.ds(off, 16)]
                sig = 1.0 / (1.0 + jnp.exp(-dt))
                v = w16 * (1.0 - 2.0 * sig)
                plsc.addupdate_scatter(acc, [sl16, zcol], v)
            return carry2

        lax.fori_loop(jnp.int32(0), jnp.int32(NIN), do_chunk, jnp.int32(0))
        return carry

    lax.fori_loop(jnp.int32(0), jnp.int32(NPH), do_phase, jnp.int32(0))
    pltpu.sync_copy(acc.at[pl.ds(0, B)], out_hbm.at[pl.ds(wid * B, B)])


@functools.lru_cache(maxsize=None)
def _make_corr_scat():
    return pl.kernel(
        _corr_scat_body,
        out_type=jax.ShapeDtypeStruct((NPAD, 16), jnp.float32),
        mesh=_mesh(),
        name="sc_corr_scat",
        compiler_params=_SC_PARAMS,
        scratch_types=[
            pltpu.VMEM((EC,), jnp.float32),
            pltpu.VMEM((NIN, CH), jnp.int32),
            pltpu.VMEM((NIN, CH), jnp.float32),
            pltpu.VMEM((ACCR, 16), jnp.float32),
        ],
    )


def _leaky(t):
    return jnp.where(t >= 0, t, 0.01 * t)


def _tca_body(x_ref, w_ref, dg_ref, q_ref, dinv_ref):
    deg = dg_ref[:, 0:1] + 1.0
    dinv = lax.rsqrt(deg)
    p = jnp.dot(x_ref[...], w_ref[...], preferred_element_type=jnp.float32)
    q_ref[...] = p * dinv
    dinv_ref[...] = dinv


def _tcb_body(s_ref, q_ref, dinv_ref, w_ref, q2_ref):
    dinv = dinv_ref[...]
    h = jnp.maximum((s_ref[...] + q_ref[...]) * dinv, 0.0)
    q2_ref[...] = jnp.dot(h, w_ref[...],
                          preferred_element_type=jnp.float32) * dinv


def _tcc_body(s_ref, q_ref, dinv_ref, qz_ref):
    dinv = dinv_ref[...]
    z = jnp.maximum((s_ref[...] + q_ref[...]) * dinv, 0.0)
    qz_ref[...] = z * dinv


def _tcd_body(s_ref, q_ref, dinv_ref, wa_ref, ws_ref, s1_ref, q4_ref):
    dinv = dinv_ref[...]
    az = (s_ref[...] + q_ref[...]) * dinv
    a1 = _leaky(jnp.dot(az, wa_ref[...], preferred_element_type=jnp.float32))
    s1_ref[...] = _leaky(jnp.dot(az, ws_ref[...],
                                 preferred_element_type=jnp.float32))
    q4_ref[...] = a1 * dinv


def _tce_body(s_ref, q_ref, dinv_ref, w_ref, x_ref, attr_ref, ae_ref):
    dinv = dinv_ref[...]
    t = (s_ref[...] + q_ref[...]) * dinv
    attr = _leaky(jnp.dot(t, w_ref[...], preferred_element_type=jnp.float32))
    attr_ref[...] = attr
    diff = x_ref[...] - attr
    ae_ref[...] = jnp.sqrt(jnp.sum(diff * diff, axis=1, keepdims=True))


def _dense_body(s1b_ref, s1f_ref, st_ref, rq_ref):
    p = lax.dot_general(s1b_ref[...], s1f_ref[...],
                        (((1,), (1,)), ((), ())),
                        preferred_element_type=jnp.float32)
    sg = jax.nn.sigmoid(p)
    st_ref[...] = sg
    rq_ref[...] = jnp.sum(sg * sg, axis=1, keepdims=True)


def _tcf_body(rq_ref, cp_ref, ae_ref, sc_ref):
    corr = cp_ref[:, 0:1]
    se = jnp.sqrt(jnp.maximum(rq_ref[...] + corr, 0.0))
    rec = 0.5 * ae_ref[...] + 0.5 * se
    mn = jnp.min(rec)
    mx = jnp.max(rec)
    sc_ref[...] = (rec - mn) / (mx - mn)


def _partition(major, minors, pads, padrow):
    bnd = jnp.searchsorted(major, (jnp.arange(NW + 1, dtype=jnp.int32) * B)
                           .astype(major.dtype)).astype(jnp.int32)
    j = jnp.arange(EC, dtype=jnp.int32)
    g = jnp.clip(bnd[:NW, None] + j[None, :], 0, E - 1)
    valid = j[None, :] < (bnd[1:, None] - bnd[:NW, None])
    majloc = jnp.where(valid, major[g] - (jnp.arange(NW, dtype=jnp.int32)
                                          * B)[:, None], padrow)
    outs = [jnp.where(valid, m[g], pv) for m, pv in zip(minors, pads)]
    perm = (j % 16) * (EC // 16) + j // 16
    majloc = majloc[:, perm]
    outs = [o[:, perm] for o in outs]
    shape = (NW, NPH, NIN, CH)
    return (majloc.astype(jnp.int32).reshape(shape),
            [o.reshape(shape) for o in outs])


def kernel(x, edge_index, W_enc1, W_enc2, W_attr1, W_attr2, W_struct1):
    with _config.enable_x64(False):
        return _kernel_impl(x, edge_index, W_enc1, W_enc2, W_attr1, W_attr2,
                            W_struct1)


def _kernel_impl(x, edge_index, W_enc1, W_enc2, W_attr1, W_attr2, W_struct1):
    x = x.astype(jnp.float32)
    ei = edge_index.astype(jnp.int32)
    src, dst = ei[0], ei[1]

    k1 = src * N + dst
    ks1 = jnp.sort(k1)
    src_s = ks1 // N
    dst_s = ks1 - src_s * N
    wdup = jnp.concatenate([
        jnp.ones((1,), jnp.float32),
        (ks1[1:] != ks1[:-1]).astype(jnp.float32),
    ])
    k2 = dst * N + src
    ks2 = jnp.sort(k2)
    dst_t = ks2 // N
    src_t = ks2 - dst_t * N

    dstloc_p, (srcg_p,) = _partition(dst_t, [src_t], [jnp.int32(0)], B)
    srcloc_p, (dstg_p, w_p) = _partition(
        src_s, [dst_s, wdup], [jnp.int32(0), jnp.float32(0.0)], 0)

    degp = _make_deg()(dstloc_p)

    q1, dinv = pl.pallas_call(
        _tca_body,
        name="tc_a",
        grid=(N // RB,),
        in_specs=[
            pl.BlockSpec((RB, 128), lambda i: (i, 0)),
            pl.BlockSpec((128, 64), lambda i: (0, 0)),
            pl.BlockSpec((RB, 16), lambda i: (i, 0)),
        ],
        out_specs=[
            pl.BlockSpec((RB, 64), lambda i: (i, 0)),
            pl.BlockSpec((RB, 1), lambda i: (i, 0)),
        ],
        out_shape=[
            jax.ShapeDtypeStruct((NPAD, 64), jnp.float32),
            jax.ShapeDtypeStruct((N, 1), jnp.float32),
        ],
    )(x, W_enc1, degp)

    s1p = _make_segsum(64)(q1.T, srcg_p, dstloc_p)

    q2 = pl.pallas_call(
        _tcb_body,
        name="tc_b",
        grid=(N // RB,),
        in_specs=[
            pl.BlockSpec((RB, 64), lambda i: (i, 0)),
            pl.BlockSpec((RB, 64), lambda i: (i, 0)),
            pl.BlockSpec((RB, 1), lambda i: (i, 0)),
            pl.BlockSpec((64, 32), lambda i: (0, 0)),
        ],
        out_specs=pl.BlockSpec((RB, 32), lambda i: (i, 0)),
        out_shape=jax.ShapeDtypeStruct((NPAD, 32), jnp.float32),
    )(s1p, q1, dinv, W_enc2)

    s2p = _make_segsum(32)(q2.T, srcg_p, dstloc_p)

    qz = pl.pallas_call(
        _tcc_body,
        name="tc_c",
        grid=(N // RB,),
        in_specs=[
            pl.BlockSpec((RB, 32), lambda i: (i, 0)),
            pl.BlockSpec((RB, 32), lambda i: (i, 0)),
            pl.BlockSpec((RB, 1), lambda i: (i, 0)),
        ],
        out_specs=pl.BlockSpec((RB, 32), lambda i: (i, 0)),
        out_shape=jax.ShapeDtypeStruct((NPAD, 32), jnp.float32),
    )(s2p, q2, dinv)

    s3p = _make_segsum(32)(qz.T, srcg_p, dstloc_p)

    s1v, q4 = pl.pallas_call(
        _tcd_body,
        name="tc_d",
        grid=(N // RB,),
        in_specs=[
            pl.BlockSpec((RB, 32), lambda i: (i, 0)),
            pl.BlockSpec((RB, 32), lambda i: (i, 0)),
            pl.BlockSpec((RB, 1), lambda i: (i, 0)),
            pl.BlockSpec((32, 64), lambda i: (0, 0)),
            pl.BlockSpec((32, 64), lambda i: (0, 0)),
        ],
        out_specs=[
            pl.BlockSpec((RB, 64), lambda i: (i, 0)),
            pl.BlockSpec((RB, 64), lambda i: (i, 0)),
        ],
        out_shape=[
            jax.ShapeDtypeStruct((NPAD, 64), jnp.float32),
            jax.ShapeDtypeStruct((NPAD, 64), jnp.float32),
        ],
    )(s3p, qz, dinv, W_attr1, W_struct1)

    s4p = _make_segsum(64)(q4.T, srcg_p, dstloc_p)

    attr, attr_err = pl.pallas_call(
        _tce_body,
        name="tc_e",
        grid=(N // RB,),
        in_specs=[
            pl.BlockSpec((RB, 64), lambda i: (i, 0)),
            pl.BlockSpec((RB, 64), lambda i: (i, 0)),
            pl.BlockSpec((RB, 1), lambda i: (i, 0)),
            pl.BlockSpec((64, 128), lambda i: (0, 0)),
            pl.BlockSpec((RB, 128), lambda i: (i, 0)),
        ],
        out_specs=[
            pl.BlockSpec((RB, 128), lambda i: (i, 0)),
            pl.BlockSpec((RB, 1), lambda i: (i, 0)),
        ],
        out_shape=[
            jax.ShapeDtypeStruct((N, 128), jnp.float32),
            jax.ShapeDtypeStruct((N, 1), jnp.float32),
        ],
    )(s4p, q4, dinv, W_attr2, x)

    struct, rowsq = pl.pallas_call(
        _dense_body,
        name="tc_dense",
        grid=(N // RBD,),
        in_specs=[
            pl.BlockSpec((RBD, 64), lambda i: (i, 0)),
            pl.BlockSpec((N, 64), lambda i: (0, 0)),
        ],
        out_specs=[
            pl.BlockSpec((RBD, N), lambda i: (i, 0)),
            pl.BlockSpec((RBD, 1), lambda i: (i, 0)),
        ],
        out_shape=[
            jax.ShapeDtypeStruct((N, N), jnp.float32),
            jax.ShapeDtypeStruct((N, 1), jnp.float32),
        ],
    )(s1v, s1v)

    dotsv = _make_corr_dots()(s1v.T, srcloc_p, dstg_p)
    corrp = _make_corr_scat()(dotsv, srcloc_p, w_p)

    scores = pl.pallas_call(
        _tcf_body,
        name="tc_f",
        grid=(1,),
        in_specs=[
            pl.BlockSpec((N, 1), lambda i: (0, 0)),
            pl.BlockSpec((N, 16), lambda i: (0, 0)),
            pl.BlockSpec((N, 1), lambda i: (0, 0)),
        ],
        out_specs=pl.BlockSpec((N, 1), lambda i: (0, 0)),
        out_shape=jax.ShapeDtypeStruct((N, 1), jnp.float32),
    )(rowsq, corrp, attr_err)

    return (attr, struct, scores.reshape(N))


# speedup vs baseline: 3.5304x; 1.1519x over previous
import functools

import jax
import jax.numpy as jnp
from jax import lax
from jax.experimental import pallas as pl
from jax.experimental.pallas import tpu as pltpu
from jax.experimental.pallas import tpu_sc as plsc
from jax._src import config as _config

N = 10000
E = 320000
NW = 32
NPAD = 10240
B = 320
ACCR = 328
EC = 10880
NPH = 5
NIN = 17
CH = 128
RB = 2000
RBD = 200


@functools.lru_cache(maxsize=None)
def _mesh():
    return plsc.VectorSubcoreMesh(core_axis_name="c", subcore_axis_name="s",
                                  num_cores=2, num_subcores=16)


_SC_PARAMS = pltpu.CompilerParams(needs_layout_passes=False)


def _wid():
    return lax.axis_index("c") * 16 + lax.axis_index("s")


def _zero_rows(ref, nrows, ncol16):
    z16 = jnp.zeros((16,), jnp.float32)

    def zrow(i, carry):
        for j in range(ncol16):
            ref[i, pl.ds(j * 16, 16)] = z16
        return carry

    lax.fori_loop(jnp.int32(0), jnp.int32(nrows), zrow, jnp.int32(0))


@functools.lru_cache(maxsize=None)
def _make_segsum(d):
    nslab = d // 8

    def body(qT_hbm, srcg_hbm, dstloc_hbm, out_hbm, slab, idx_s, idx_d, acc):
        wid = _wid()
        _zero_rows(acc, ACCR, d // 16)

        def do_slab(s, carry):
            pltpu.sync_copy(qT_hbm.at[pl.ds(s * 8, 8)], slab)
            rowvs = [jnp.full((16,), col, jnp.int32) for col in range(8)]
            colvs = [(s * 8 + col) + jnp.zeros((16,), jnp.int32)
                     for col in range(8)]

            def do_phase(p, carry2):
                pltpu.sync_copy(srcg_hbm.at[wid, p], idx_s)
                pltpu.sync_copy(dstloc_hbm.at[wid, p], idx_d)

                def do_chunk(r, carry3):
                    for g in range(CH // 16):
                        src16 = idx_s[r, pl.ds(g * 16, 16)]
                        dl16 = idx_d[r, pl.ds(g * 16, 16)]
                        for col in range(8):
                            v = plsc.load_gather(slab, [rowvs[col], src16])
                            plsc.addupdate_scatter(acc, [dl16, colvs[col]], v)
                    return carry3

                lax.fori_loop(jnp.int32(0), jnp.int32(NIN), do_chunk,
                              jnp.int32(0))
                return carry2

            lax.fori_loop(jnp.int32(0), jnp.int32(NPH), do_phase, jnp.int32(0))
            return carry

        lax.fori_loop(jnp.int32(0), jnp.int32(nslab), do_slab, jnp.int32(0))
        pltpu.sync_copy(acc.at[pl.ds(0, B)], out_hbm.at[pl.ds(wid * B, B)])

    return pl.kernel(
        body,
        out_type=jax.ShapeDtypeStruct((NPAD, d), jnp.float32),
        mesh=_mesh(),
        name=f"sc_segsum{d}",
        compiler_params=_SC_PARAMS,
        scratch_types=[
            pltpu.VMEM((8, NPAD), jnp.float32),
            pltpu.VMEM((NIN, CH), jnp.int32),
            pltpu.VMEM((NIN, CH), jnp.int32),
            pltpu.VMEM((ACCR, d), jnp.float32),
        ],
    )


def _deg_body(dstloc_hbm, out_hbm, idx_d, acc):
    wid = _wid()
    _zero_rows(acc, ACCR, 1)
    ones16 = jnp.ones((16,), jnp.float32)
    zcol = jnp.zeros((16,), jnp.int32)

    def do_phase(p, carry):
        pltpu.sync_copy(dstloc_hbm.at[wid, p], idx_d)

        def do_chunk(r, carry2):
            for g in range(CH // 16):
                dl16 = idx_d[r, pl.ds(g * 16, 16)]
                plsc.addupdate_scatter(acc, [dl16, zcol], ones16)
            return carry2

        lax.fori_loop(jnp.int32(0), jnp.int32(NIN), do_chunk, jnp.int32(0))
        return carry

    lax.fori_loop(jnp.int32(0), jnp.int32(NPH), do_phase, jnp.int32(0))
    pltpu.sync_copy(acc.at[pl.ds(0, B)], out_hbm.at[pl.ds(wid * B, B)])


@functools.lru_cache(maxsize=None)
def _make_deg():
    return pl.kernel(
        _deg_body,
        out_type=jax.ShapeDtypeStruct((NPAD, 16), jnp.float32),
        mesh=_mesh(),
        name="sc_deg",
        compiler_params=_SC_PARAMS,
        scratch_types=[
            pltpu.VMEM((NIN, CH), jnp.int32),
            pltpu.VMEM((ACCR, 16), jnp.float32),
        ],
    )


def _corr_dots_body(s1T_hbm, srcloc_hbm, dstg_hbm, out_hbm,
                    slab, idx_a, idx_b, dots):
    wid = _wid()
    z16 = jnp.zeros((16,), jnp.float32)

    def zdots(i, carry):
        dots[pl.ds(i * 16, 16)] = z16
        return carry

    lax.fori_loop(jnp.int32(0), jnp.int32(EC // 16), zdots, jnp.int32(0))
    base = wid * B

    def do_slab(s, carry):
        pltpu.sync_copy(s1T_hbm.at[pl.ds(s * 8, 8)], slab)
        rowvs = [jnp.full((16,), col, jnp.int32) for col in range(8)]

        def do_phase(p, carry2):
            pltpu.sync_copy(srcloc_hbm.at[wid, p], idx_a)
            pltpu.sync_copy(dstg_hbm.at[wid, p], idx_b)

            def do_chunk(r, carry3):
                for g in range(CH // 16):
                    srcg16 = idx_a[r, pl.ds(g * 16, 16)] + base
                    dst16 = idx_b[r, pl.ds(g * 16, 16)]
                    part = jnp.zeros((16,), jnp.float32)
                    for col in range(8):
                        a = plsc.load_gather(slab, [rowvs[col], srcg16])
                        b = plsc.load_gather(slab, [rowvs[col], dst16])
                        part = part + a * b
                    off = (p * NIN + r) * CH + g * 16
                    dots[pl.ds(off, 16)] = dots[pl.ds(off, 16)] + part
                return carry3

            lax.fori_loop(jnp.int32(0), jnp.int32(NIN), do_chunk, jnp.int32(0))
            return carry2

        lax.fori_loop(jnp.int32(0), jnp.int32(NPH), do_phase, jnp.int32(0))
        return carry

    lax.fori_loop(jnp.int32(0), jnp.int32(8), do_slab, jnp.int32(0))
    pltpu.sync_copy(dots, out_hbm.at[wid])


@functools.lru_cache(maxsize=None)
def _make_corr_dots():
    return pl.kernel(
        _corr_dots_body,
        out_type=jax.ShapeDtypeStruct((NW, EC), jnp.float32),
        mesh=_mesh(),
        name="sc_corr_dots",
        compiler_params=_SC_PARAMS,
        scratch_types=[
            pltpu.VMEM((8, NPAD), jnp.float32),
            pltpu.VMEM((NIN, CH), jnp.int32),
            pltpu.VMEM((NIN, CH), jnp.int32),
            pltpu.VMEM((EC,), jnp.float32),
        ],
    )


def _corr_scat_body(dots_hbm, srcloc_hbm, w_hbm, out_hbm, dots, idx_a, wbuf,
                    acc):
    wid = _wid()
    _zero_rows(acc, ACCR, 1)
    zcol = jnp.zeros((16,), jnp.int32)
    pltpu.sync_copy(dots_hbm.at[wid], dots)

    def do_phase(p, carry):
        pltpu.sync_copy(srcloc_hbm.at[wid, p], idx_a)
        pltpu.sync_copy(w_hbm.at[wid, p], wbuf)

        def do_chunk(r, carry2):
            for g in range(CH // 16):
                sl16 = idx_a[r, pl.ds(g * 16, 16)]
                w16 = wbuf[r, pl.ds(g * 16, 16)]
                off = (p * NIN + r) * CH + g * 16
                dt = dots[pl.ds(off, 16)]
                sig = 1.0 / (1.0 + jnp.exp(-dt))
                v = w16 * (1.0 - 2.0 * sig)
                plsc.addupdate_scatter(acc, [sl16, zcol], v)
            return carry2

        lax.fori_loop(jnp.int32(0), jnp.int32(NIN), do_chunk, jnp.int32(0))
        return carry

    lax.fori_loop(jnp.int32(0), jnp.int32(NPH), do_phase, jnp.int32(0))
    pltpu.sync_copy(acc.at[pl.ds(0, B)], out_hbm.at[pl.ds(wid * B, B)])


@functools.lru_cache(maxsize=None)
def _make_corr_scat():
    return pl.kernel(
        _corr_scat_body,
        out_type=jax.ShapeDtypeStruct((NPAD, 16), jnp.float32),
        mesh=_mesh(),
        name="sc_corr_scat",
        compiler_params=_SC_PARAMS,
        scratch_types=[
            pltpu.VMEM((EC,), jnp.float32),
            pltpu.VMEM((NIN, CH), jnp.int32),
            pltpu.VMEM((NIN, CH), jnp.float32),
            pltpu.VMEM((ACCR, 16), jnp.float32),
        ],
    )


def _leaky(t):
    return jnp.where(t >= 0, t, 0.01 * t)


def _tca_body(x_ref, w_ref, dg_ref, q_ref, dinv_ref):
    deg = dg_ref[:, 0:1] + 1.0
    dinv = lax.rsqrt(deg)
    p = jnp.dot(x_ref[...], w_ref[...], preferred_element_type=jnp.float32)
    q_ref[...] = p * dinv
    dinv_ref[...] = dinv


def _tcb_body(s_ref, q_ref, dinv_ref, w_ref, q2_ref):
    dinv = dinv_ref[...]
    h = jnp.maximum((s_ref[...] + q_ref[...]) * dinv, 0.0)
    q2_ref[...] = jnp.dot(h, w_ref[...],
                          preferred_element_type=jnp.float32) * dinv


def _tcc_body(s_ref, q_ref, dinv_ref, qz_ref):
    dinv = dinv_ref[...]
    z = jnp.maximum((s_ref[...] + q_ref[...]) * dinv, 0.0)
    qz_ref[...] = z * dinv


def _tcd_body(s_ref, q_ref, dinv_ref, wa_ref, ws_ref, s1_ref, q4_ref):
    dinv = dinv_ref[...]
    az = (s_ref[...] + q_ref[...]) * dinv
    a1 = _leaky(jnp.dot(az, wa_ref[...], preferred_element_type=jnp.float32))
    s1_ref[...] = _leaky(jnp.dot(az, ws_ref[...],
                                 preferred_element_type=jnp.float32))
    q4_ref[...] = a1 * dinv


def _tce_body(s_ref, q_ref, dinv_ref, w_ref, x_ref, attr_ref, ae_ref):
    dinv = dinv_ref[...]
    t = (s_ref[...] + q_ref[...]) * dinv
    attr = _leaky(jnp.dot(t, w_ref[...], preferred_element_type=jnp.float32))
    attr_ref[...] = attr
    diff = x_ref[...] - attr
    ae_ref[...] = jnp.sqrt(jnp.sum(diff * diff, axis=1, keepdims=True))


def _dense_body(s1b_ref, s1f_ref, st_ref, rq_ref):
    p = lax.dot_general(s1b_ref[...], s1f_ref[...],
                        (((1,), (1,)), ((), ())),
                        preferred_element_type=jnp.float32)
    sg = jax.nn.sigmoid(p)
    st_ref[...] = sg
    rq_ref[...] = jnp.sum(sg * sg, axis=1, keepdims=True)


def _tcf_body(rq_ref, cp_ref, ae_ref, sc_ref):
    corr = cp_ref[:, 0:1]
    se = jnp.sqrt(jnp.maximum(rq_ref[...] + corr, 0.0))
    rec = 0.5 * ae_ref[...] + 0.5 * se
    mn = jnp.min(rec)
    mx = jnp.max(rec)
    sc_ref[...] = (rec - mn) / (mx - mn)


def _partition(major, minors, pads, padrow):
    bnd = jnp.searchsorted(major, (jnp.arange(NW + 1, dtype=jnp.int32) * B)
                           .astype(major.dtype)).astype(jnp.int32)
    j = jnp.arange(EC, dtype=jnp.int32)
    g = jnp.clip(bnd[:NW, None] + j[None, :], 0, E - 1)
    valid = j[None, :] < (bnd[1:, None] - bnd[:NW, None])
    majloc = jnp.where(valid, major[g] - (jnp.arange(NW, dtype=jnp.int32)
                                          * B)[:, None], padrow)
    outs = [jnp.where(valid, m[g], pv) for m, pv in zip(minors, pads)]
    perm = (j % 16) * (EC // 16) + j // 16
    majloc = majloc[:, perm]
    outs = [o[:, perm] for o in outs]
    shape = (NW, NPH, NIN, CH)
    return (majloc.astype(jnp.int32).reshape(shape),
            [o.reshape(shape) for o in outs])


def kernel(x, edge_index, W_enc1, W_enc2, W_attr1, W_attr2, W_struct1):
    with _config.enable_x64(False):
        return _kernel_impl(x, edge_index, W_enc1, W_enc2, W_attr1, W_attr2,
                            W_struct1)


def _kernel_impl(x, edge_index, W_enc1, W_enc2, W_attr1, W_attr2, W_struct1):
    x = x.astype(jnp.float32)
    ei = edge_index.astype(jnp.int32)
    src, dst = ei[0], ei[1]

    k1 = src * N + dst
    ks1 = jnp.sort(k1)
    src_s = ks1 // N
    dst_s = ks1 - src_s * N
    wdup = jnp.concatenate([
        jnp.ones((1,), jnp.float32),
        (ks1[1:] != ks1[:-1]).astype(jnp.float32),
    ])
    k2 = dst * N + src
    ks2 = jnp.sort(k2)
    dst_t = ks2 // N
    src_t = ks2 - dst_t * N

    dstloc_p, (srcg_p,) = _partition(dst_t, [src_t], [jnp.int32(0)], B)
    srcloc_p, (dstg_p, w_p) = _partition(
        src_s, [dst_s, wdup], [jnp.int32(0), jnp.float32(0.0)], 0)

    degp = _make_deg()(dstloc_p)

    q1, dinv = pl.pallas_call(
        _tca_body,
        name="tc_a",
        grid=(N // RB,),
        in_specs=[
            pl.BlockSpec((RB, 128), lambda i: (i, 0)),
            pl.BlockSpec((128, 64), lambda i: (0, 0)),
            pl.BlockSpec((RB, 16), lambda i: (i, 0)),
        ],
        out_specs=[
            pl.BlockSpec((RB, 64), lambda i: (i, 0)),
            pl.BlockSpec((RB, 1), lambda i: (i, 0)),
        ],
        out_shape=[
            jax.ShapeDtypeStruct((NPAD, 64), jnp.float32),
            jax.ShapeDtypeStruct((N, 1), jnp.float32),
        ],
    )(x, W_enc1, degp)

    s1p = _make_segsum(64)(q1.T, srcg_p, dstloc_p)

    q2 = pl.pallas_call(
        _tcb_body,
        name="tc_b",
        grid=(N // RB,),
        in_specs=[
            pl.BlockSpec((RB, 64), lambda i: (i, 0)),
            pl.BlockSpec((RB, 64), lambda i: (i, 0)),
            pl.BlockSpec((RB, 1), lambda i: (i, 0)),
            pl.BlockSpec((64, 32), lambda i: (0, 0)),
        ],
        out_specs=pl.BlockSpec((RB, 32), lambda i: (i, 0)),
        out_shape=jax.ShapeDtypeStruct((NPAD, 32), jnp.float32),
    )(s1p, q1, dinv, W_enc2)

    s2p = _make_segsum(32)(q2.T, srcg_p, dstloc_p)

    qz = pl.pallas_call(
        _tcc_body,
        name="tc_c",
        grid=(N // RB,),
        in_specs=[
            pl.BlockSpec((RB, 32), lambda i: (i, 0)),
            pl.BlockSpec((RB, 32), lambda i: (i, 0)),
            pl.BlockSpec((RB, 1), lambda i: (i, 0)),
        ],
        out_specs=pl.BlockSpec((RB, 32), lambda i: (i, 0)),
        out_shape=jax.ShapeDtypeStruct((NPAD, 32), jnp.float32),
    )(s2p, q2, dinv)

    s3p = _make_segsum(32)(qz.T, srcg_p, dstloc_p)

    s1v, q4 = pl.pallas_call(
        _tcd_body,
        name="tc_d",
        grid=(N // RB,),
        in_specs=[
            pl.BlockSpec((RB, 32), lambda i: (i, 0)),
            pl.BlockSpec((RB, 32), lambda i: (i, 0)),
            pl.BlockSpec((RB, 1), lambda i: (i, 0)),
            pl.BlockSpec((32, 64), lambda i: (0, 0)),
            pl.BlockSpec((32, 64), lambda i: (0, 0)),
        ],
        out_specs=[
            pl.BlockSpec((RB, 64), lambda i: (i, 0)),
            pl.BlockSpec((RB, 64), lambda i: (i, 0)),
        ],
        out_shape=[
            jax.ShapeDtypeStruct((NPAD, 64), jnp.float32),
            jax.ShapeDtypeStruct((NPAD, 64), jnp.float32),
        ],
    )(s3p, qz, dinv, W_attr1, W_struct1)

    s4p = _make_segsum(64)(q4.T, srcg_p, dstloc_p)

    attr, attr_err = pl.pallas_call(
        _tce_body,
        name="tc_e",
        grid=(N // RB,),
        in_specs=[
            pl.BlockSpec((RB, 64), lambda i: (i, 0)),
            pl.BlockSpec((RB, 64), lambda i: (i, 0)),
            pl.BlockSpec((RB, 1), lambda i: (i, 0)),
            pl.BlockSpec((64, 128), lambda i: (0, 0)),
            pl.BlockSpec((RB, 128), lambda i: (i, 0)),
        ],
        out_specs=[
            pl.BlockSpec((RB, 128), lambda i: (i, 0)),
            pl.BlockSpec((RB, 1), lambda i: (i, 0)),
        ],
        out_shape=[
            jax.ShapeDtypeStruct((N, 128), jnp.float32),
            jax.ShapeDtypeStruct((N, 1), jnp.float32),
        ],
    )(s4p, q4, dinv, W_attr2, x)

    struct, rowsq = pl.pallas_call(
        _dense_body,
        name="tc_dense",
        grid=(N // RBD,),
        in_specs=[
            pl.BlockSpec((RBD, 64), lambda i: (i, 0)),
            pl.BlockSpec((N, 64), lambda i: (0, 0)),
        ],
        out_specs=[
            pl.BlockSpec((RBD, N), lambda i: (i, 0)),
            pl.BlockSpec((RBD, 1), lambda i: (i, 0)),
        ],
        out_shape=[
            jax.ShapeDtypeStruct((N, N), jnp.float32),
            jax.ShapeDtypeStruct((N, 1), jnp.float32),
        ],
    )(s1v, s1v)

    dotsv = _make_corr_dots()(s1v.T, srcloc_p, dstg_p)
    corrp = _make_corr_scat()(dotsv, srcloc_p, w_p)

    scores = pl.pallas_call(
        _tcf_body,
        name="tc_f",
        grid=(1,),
        in_specs=[
            pl.BlockSpec((N, 1), lambda i: (0, 0)),
            pl.BlockSpec((N, 16), lambda i: (0, 0)),
            pl.BlockSpec((N, 1), lambda i: (0, 0)),
        ],
        out_specs=pl.BlockSpec((N, 1), lambda i: (0, 0)),
        out_shape=jax.ShapeDtypeStruct((N, 1), jnp.float32),
    )(rowsq, corrp, attr_err)

    return (attr, struct, scores.reshape(N))
